# baseline (speedup 1.0000x reference)
import functools

import jax
import jax.numpy as jnp
from jax import lax
from jax.experimental import pallas as pl
from jax.experimental.pallas import tpu as pltpu
from jax.experimental.pallas import tpu_sc as plsc

_DIM = 16
_G = 128
_CHUNK = 1024


@functools.lru_cache(maxsize=None)
def _build_gather(b_batch: int, l_seq: int, vocab: int):
    info = plsc.get_sparse_core_info()
    nc, ns = info.num_cores, info.num_subcores
    nw = nc * ns
    total = b_batch * l_seq
    npw = total // nw
    assert npw * nw == total and npw % _CHUNK == 0
    ng = npw // _G
    gpc = _CHUNK // _G

    mesh = plsc.VectorSubcoreMesh(core_axis_name="c", subcore_axis_name="s")

    @functools.partial(
        pl.kernel,
        mesh=mesh,
        compiler_params=pltpu.CompilerParams(use_tc_tiling_on_sc=True),
        out_type=[
            jax.ShapeDtypeStruct((nw, npw * _DIM), jnp.float32),
            jax.ShapeDtypeStruct((nw, npw * _DIM), jnp.float32),
        ],
        scratch_types=[
            pltpu.VMEM((ng, _G), jnp.int32),
            pltpu.VMEM((2, _G), jnp.int32),
            pltpu.VMEM((2, _G, 128), jnp.float32),
            pltpu.VMEM((2, _G, 128), jnp.float32),
            pltpu.VMEM((_CHUNK * _DIM,), jnp.float32),
            pltpu.VMEM((_CHUNK * _DIM,), jnp.float32),
            pltpu.SemaphoreType.DMA,
            pltpu.SemaphoreType.DMA,
        ],
    )
    def gather_kernel(ids_hbm, er_hbm, ei_hbm, out_r, out_i,
                      idx_v, blk_idx, blk_r, blk_i, pr_v, pi_v,
                      sem_r, sem_i):
        wid = lax.axis_index("s") * nc + lax.axis_index("c")
        pltpu.sync_copy(ids_hbm.at[wid], idx_v)

        def stage_and_fire(g):
            slot = lax.rem(g, 2)
            for k in range(_G // 16):
                sl = pl.ds(k * 16, 16)
                blk_idx[slot, sl] = lax.shift_right_logical(idx_v[g, sl], 3)
            pltpu.make_async_copy(
                er_hbm.at[blk_idx.at[slot]], blk_r.at[slot], sem_r).start()
            pltpu.make_async_copy(
                ei_hbm.at[blk_idx.at[slot]], blk_i.at[slot], sem_i).start()

        stage_and_fire(0)

        def body(g, carry):
            slot = lax.rem(g, 2)

            @pl.when(g + 1 < ng)
            def _():
                stage_and_fire(g + 1)

            pltpu.make_async_copy(
                er_hbm.at[blk_idx.at[slot]], blk_r.at[slot], sem_r).wait()
            pltpu.make_async_copy(
                ei_hbm.at[blk_idx.at[slot]], blk_i.at[slot], sem_i).wait()

            base = lax.rem(g, gpc) * _G
            for t in range(_G // 16):
                idvec = idx_v[g, pl.ds(t * 16, 16)]
                col0v = (idvec & 7) * _DIM
                for lane in range(16):
                    i = t * 16 + lane
                    col0 = col0v[lane]
                    out_sl = pl.ds((base + i) * _DIM, _DIM)
                    pr_v[out_sl] = blk_r[slot, i, pl.ds(col0, _DIM)]
                    pi_v[out_sl] = blk_i[slot, i, pl.ds(col0, _DIM)]

            @pl.when(lax.rem(g, gpc) == gpc - 1)
            def _():
                t0 = (g // gpc) * _CHUNK * _DIM
                pltpu.sync_copy(pr_v, out_r.at[wid, pl.ds(t0, _CHUNK * _DIM)])
                pltpu.sync_copy(pi_v, out_i.at[wid, pl.ds(t0, _CHUNK * _DIM)])

            return carry

        lax.fori_loop(0, ng, body, 0)

    return gather_kernel, nw, ng


def kernel(ids, embed, imag_embed):
    b, l = ids.shape
    vocab = embed.shape[0]
    gather_kernel, nw, ng = _build_gather(b, l, vocab)
    ids_u = ids.T.reshape(nw, ng, _G).astype(jnp.int32)
    er = embed.reshape(vocab // 8, 128)
    ei = imag_embed.reshape(vocab // 8, 128)
    out_r, out_i = gather_kernel(ids_u, er, ei)
    r_t = lax.transpose(out_r.reshape(l, b, _DIM), (0, 2, 1))
    i_t = lax.transpose(out_i.reshape(l, b, _DIM), (0, 2, 1))
    r_t, i_t = lax.optimization_barrier((r_t, i_t))
    c_t = lax.optimization_barrier(lax.complex(r_t, i_t))
    return lax.transpose(c_t, (2, 0, 1))


# speedup vs baseline: 2.8002x; 2.8002x over previous
import functools

import jax
import jax.numpy as jnp
from jax import lax
from jax.experimental import pallas as pl
from jax.experimental.pallas import tpu as pltpu
from jax.experimental.pallas import tpu_sc as plsc

_DIM = 16
_G = 128
_CH = 8


@functools.lru_cache(maxsize=None)
def _build_gather(b_batch: int, l_seq: int, vocab: int):
    info = plsc.get_sparse_core_info()
    nc, ns = info.num_cores, info.num_subcores
    nw = nc * ns
    total = b_batch * l_seq
    npw = total // nw
    rows = _CH * _G
    assert npw * nw == total and npw % rows == 0
    assert b_batch % rows == 0
    ng = npw // _G
    nchunk = ng // _CH

    mesh = plsc.VectorSubcoreMesh(core_axis_name="c", subcore_axis_name="s")

    @functools.partial(
        pl.kernel,
        mesh=mesh,
        compiler_params=pltpu.CompilerParams(
            use_tc_tiling_on_sc=False, needs_layout_passes=False),
        out_type=[
            jax.ShapeDtypeStruct((l_seq * _DIM * b_batch,), jnp.float32),
            jax.ShapeDtypeStruct((l_seq * _DIM * b_batch,), jnp.float32),
        ],
        scratch_types=[
            pltpu.VMEM((ng, _G), jnp.int32),
            pltpu.VMEM((rows, _DIM), jnp.float32),
            pltpu.VMEM((rows, _DIM), jnp.float32),
            pltpu.VMEM((_DIM * rows,), jnp.float32),
            pltpu.VMEM((_DIM * rows,), jnp.float32),
            pltpu.SemaphoreType.DMA,
            pltpu.SemaphoreType.DMA,
            pltpu.SemaphoreType.DMA,
        ],
    )
    def gather_kernel(ids_hbm, embed_hbm, imag_hbm, out_r, out_i,
                      idx_v, real_v, imag_v, pr_v, pi_v,
                      sem_r, sem_i, sem_o):
        wid = lax.axis_index("s") * nc + lax.axis_index("c")
        pltpu.sync_copy(ids_hbm.at[wid], idx_v)
        col16 = lax.iota(jnp.int32, 16) * rows

        def chunk_body(c, carry):
            waits = []
            for j in range(_CH):
                g = c * _CH + j
                dst = pl.ds(j * _G, _G)
                waits.append(pltpu.async_copy(
                    embed_hbm.at[idx_v.at[g]], real_v.at[dst], sem_r))
                waits.append(pltpu.async_copy(
                    imag_hbm.at[idx_v.at[g]], imag_v.at[dst], sem_i))
            for w in waits:
                w.wait()

            def tok_body(tok, carry2):
                cols = col16 + tok
                plsc.store_scatter(pr_v, [cols], real_v[tok, :])
                plsc.store_scatter(pi_v, [cols], imag_v[tok, :])
                return carry2

            lax.fori_loop(0, rows, tok_body, 0)

            u0 = wid * npw + c * rows
            l_idx = u0 // b_batch
            b0 = lax.rem(u0, b_batch)
            owaits = []
            for d in range(_DIM):
                src = pl.ds(d * rows, rows)
                o = (l_idx * _DIM + d) * b_batch + b0
                owaits.append(pltpu.async_copy(
                    pr_v.at[src], out_r.at[pl.ds(o, rows)], sem_o))
                owaits.append(pltpu.async_copy(
                    pi_v.at[src], out_i.at[pl.ds(o, rows)], sem_o))
            for w in owaits:
                w.wait()
            return carry

        lax.fori_loop(0, nchunk, chunk_body, 0)

    return gather_kernel, nw, ng


def kernel(ids, embed, imag_embed):
    b, l = ids.shape
    vocab = embed.shape[0]
    gather_kernel, nw, ng = _build_gather(b, l, vocab)
    ids_u = ids.T.reshape(nw, ng, _G).astype(jnp.int32)
    out_r, out_i = gather_kernel(ids_u, embed, imag_embed)
    r_t, i_t = lax.optimization_barrier(
        (out_r.reshape(l, _DIM, b), out_i.reshape(l, _DIM, b)))
    c_t = lax.optimization_barrier(lax.complex(r_t, i_t))
    return lax.transpose(c_t, (2, 0, 1))


# speedup vs baseline: 2.8490x; 1.0174x over previous
import functools

import jax
import jax.numpy as jnp
from jax import lax
from jax.experimental import pallas as pl
from jax.experimental.pallas import tpu as pltpu
from jax.experimental.pallas import tpu_sc as plsc

_DIM = 16
_G = 128
_CH = 8


@functools.lru_cache(maxsize=None)
def _build_gather(b_batch: int, l_seq: int, vocab: int):
    info = plsc.get_sparse_core_info()
    nc, ns = info.num_cores, info.num_subcores
    nw = nc * ns
    total = b_batch * l_seq
    npw = total // nw
    rows = _CH * _G
    assert npw * nw == total and npw % rows == 0
    assert b_batch % rows == 0
    ng = npw // _G
    nchunk = ng // _CH

    mesh = plsc.VectorSubcoreMesh(core_axis_name="c", subcore_axis_name="s")

    @functools.partial(
        pl.kernel,
        mesh=mesh,
        compiler_params=pltpu.CompilerParams(
            use_tc_tiling_on_sc=False, needs_layout_passes=False),
        out_type=[
            jax.ShapeDtypeStruct((l_seq * _DIM * b_batch,), jnp.float32),
            jax.ShapeDtypeStruct((l_seq * _DIM * b_batch,), jnp.float32),
        ],
        scratch_types=[
            pltpu.VMEM((ng, _G), jnp.int32),
            pltpu.VMEM((2, rows, _DIM), jnp.float32),
            pltpu.VMEM((2, rows, _DIM), jnp.float32),
            pltpu.VMEM((_DIM * rows,), jnp.float32),
            pltpu.VMEM((_DIM * rows,), jnp.float32),
            pltpu.SemaphoreType.DMA,
            pltpu.SemaphoreType.DMA,
            pltpu.SemaphoreType.DMA,
        ],
    )
    def gather_kernel(ids_hbm, embed_hbm, imag_hbm, out_r, out_i,
                      idx_v, real_v, imag_v, pr_v, pi_v,
                      sem_r, sem_i, sem_o):
        wid = lax.axis_index("s") * nc + lax.axis_index("c")
        pltpu.sync_copy(ids_hbm.at[wid], idx_v)
        col16 = lax.iota(jnp.int32, 16) * rows

        def fire(c, slot):
            for j in range(_CH):
                g = c * _CH + j
                dst = pl.ds(j * _G, _G)
                pltpu.make_async_copy(
                    embed_hbm.at[idx_v.at[g]],
                    real_v.at[slot].at[dst], sem_r).start()
                pltpu.make_async_copy(
                    imag_hbm.at[idx_v.at[g]],
                    imag_v.at[slot].at[dst], sem_i).start()

        def drain(c, slot):
            for j in range(_CH):
                g = c * _CH + j
                dst = pl.ds(j * _G, _G)
                pltpu.make_async_copy(
                    embed_hbm.at[idx_v.at[g]],
                    real_v.at[slot].at[dst], sem_r).wait()
                pltpu.make_async_copy(
                    imag_hbm.at[idx_v.at[g]],
                    imag_v.at[slot].at[dst], sem_i).wait()

        fire(0, 0)

        def chunk_body(c, carry):
            slot = lax.rem(c, 2)

            @pl.when(c + 1 < nchunk)
            def _():
                fire(c + 1, 1 - slot)

            drain(c, slot)

            def blk_body(blk, carry2):
                t0 = blk * 16
                for u in range(16):
                    cols = col16 + (t0 + u)
                    plsc.store_scatter(pr_v, [cols], real_v[slot, t0 + u, :])
                    plsc.store_scatter(pi_v, [cols], imag_v[slot, t0 + u, :])
                return carry2

            lax.fori_loop(0, rows // 16, blk_body, 0)

            u0 = wid * npw + c * rows
            l_idx = u0 // b_batch
            b0 = lax.rem(u0, b_batch)
            owaits = []
            for d in range(_DIM):
                src = pl.ds(d * rows, rows)
                o = (l_idx * _DIM + d) * b_batch + b0
                owaits.append(pltpu.async_copy(
                    pr_v.at[src], out_r.at[pl.ds(o, rows)], sem_o))
                owaits.append(pltpu.async_copy(
                    pi_v.at[src], out_i.at[pl.ds(o, rows)], sem_o))
            for w in owaits:
                w.wait()
            return carry

        lax.fori_loop(0, nchunk, chunk_body, 0)

    return gather_kernel, nw, ng


def kernel(ids, embed, imag_embed):
    b, l = ids.shape
    vocab = embed.shape[0]
    gather_kernel, nw, ng = _build_gather(b, l, vocab)
    ids_u = ids.T.reshape(nw, ng, _G).astype(jnp.int32)
    out_r, out_i = gather_kernel(ids_u, embed, imag_embed)
    r_t, i_t = lax.optimization_barrier(
        (out_r.reshape(l, _DIM, b), out_i.reshape(l, _DIM, b)))
    c_t = lax.optimization_barrier(lax.complex(r_t, i_t))
    return lax.transpose(c_t, (2, 0, 1))


# speedup vs baseline: 3.1344x; 1.1002x over previous
import functools

import jax
import jax.numpy as jnp
from jax import lax
from jax.experimental import pallas as pl
from jax.experimental.pallas import tpu as pltpu
from jax.experimental.pallas import tpu_sc as plsc

_DIM = 16
_G = 128
_CH = 8


@functools.lru_cache(maxsize=None)
def _build_gather(b_batch: int, l_seq: int, vocab: int):
    info = plsc.get_sparse_core_info()
    nc, ns = info.num_cores, info.num_subcores
    nw = nc * ns
    total = b_batch * l_seq
    npw = total // nw
    rows = _CH * _G
    assert npw * nw == total and npw % rows == 0
    assert b_batch % rows == 0
    ng = npw // _G
    nchunk = ng // _CH

    mesh = plsc.VectorSubcoreMesh(core_axis_name="c", subcore_axis_name="s")

    @functools.partial(
        pl.kernel,
        mesh=mesh,
        compiler_params=pltpu.CompilerParams(
            use_tc_tiling_on_sc=False, needs_layout_passes=False),
        out_type=[
            jax.ShapeDtypeStruct((l_seq * _DIM * b_batch,), jnp.float32),
            jax.ShapeDtypeStruct((l_seq * _DIM * b_batch,), jnp.float32),
        ],
        scratch_types=[
            pltpu.VMEM((ng, _G), jnp.int32),
            pltpu.VMEM((2, rows, _DIM), jnp.float32),
            pltpu.VMEM((2, rows, _DIM), jnp.float32),
            pltpu.VMEM((_DIM * (rows + 8),), jnp.float32),
            pltpu.VMEM((_DIM * (rows + 8),), jnp.float32),
            pltpu.SemaphoreType.DMA,
            pltpu.SemaphoreType.DMA,
            pltpu.SemaphoreType.DMA,
        ],
    )
    def gather_kernel(ids_hbm, embed_hbm, imag_hbm, out_r, out_i,
                      idx_v, real_v, imag_v, pr_v, pi_v,
                      sem_r, sem_i, sem_o):
        wid = lax.axis_index("s") * nc + lax.axis_index("c")
        pltpu.sync_copy(ids_hbm.at[wid], idx_v)
        col16 = lax.iota(jnp.int32, 16) * (rows + 8)

        def fire(c, slot):
            for j in range(_CH):
                g = c * _CH + j
                dst = pl.ds(j * _G, _G)
                pltpu.make_async_copy(
                    embed_hbm.at[idx_v.at[g]],
                    real_v.at[slot].at[dst], sem_r).start()
                pltpu.make_async_copy(
                    imag_hbm.at[idx_v.at[g]],
                    imag_v.at[slot].at[dst], sem_i).start()

        def drain(c, slot):
            for j in range(_CH):
                g = c * _CH + j
                dst = pl.ds(j * _G, _G)
                pltpu.make_async_copy(
                    embed_hbm.at[idx_v.at[g]],
                    real_v.at[slot].at[dst], sem_r).wait()
                pltpu.make_async_copy(
                    imag_hbm.at[idx_v.at[g]],
                    imag_v.at[slot].at[dst], sem_i).wait()

        fire(0, 0)

        def chunk_body(c, carry):
            slot = lax.rem(c, 2)

            @pl.when(c + 1 < nchunk)
            def _():
                fire(c + 1, 1 - slot)

            drain(c, slot)

            def blk_body(blk, carry2):
                t0 = blk * 16
                for u in range(16):
                    cols = col16 + (t0 + u)
                    plsc.store_scatter(pr_v, [cols], real_v[slot, t0 + u, :])
                    plsc.store_scatter(pi_v, [cols], imag_v[slot, t0 + u, :])
                return carry2

            lax.fori_loop(0, rows // 16, blk_body, 0)

            u0 = wid * npw + c * rows
            l_idx = u0 // b_batch
            b0 = lax.rem(u0, b_batch)
            owaits = []
            for d in range(_DIM):
                src = pl.ds(d * (rows + 8), rows)
                o = (l_idx * _DIM + d) * b_batch + b0
                owaits.append(pltpu.async_copy(
                    pr_v.at[src], out_r.at[pl.ds(o, rows)], sem_o))
                owaits.append(pltpu.async_copy(
                    pi_v.at[src], out_i.at[pl.ds(o, rows)], sem_o))
            for w in owaits:
                w.wait()
            return carry

        lax.fori_loop(0, nchunk, chunk_body, 0)

    return gather_kernel, nw, ng


def kernel(ids, embed, imag_embed):
    b, l = ids.shape
    vocab = embed.shape[0]
    gather_kernel, nw, ng = _build_gather(b, l, vocab)
    ids_u = ids.T.reshape(nw, ng, _G).astype(jnp.int32)
    out_r, out_i = gather_kernel(ids_u, embed, imag_embed)
    r_t, i_t = lax.optimization_barrier(
        (out_r.reshape(l, _DIM, b), out_i.reshape(l, _DIM, b)))
    c_t = lax.optimization_barrier(lax.complex(r_t, i_t))
    return lax.transpose(c_t, (2, 0, 1))
